# baseline (speedup 1.0000x reference)
import jax
import jax.numpy as jnp
from jax.experimental import pallas as pl
from jax.experimental.pallas import tpu as pltpu

_N_CHUNKS = 8


def _dma_body(pe_ref, o_ref, sems):
    rows = pe_ref.shape[0]
    chunk = rows // _N_CHUNKS
    copies = [
        pltpu.make_async_copy(
            pe_ref.at[pl.ds(i * chunk, chunk)],
            o_ref.at[pl.ds(i * chunk, chunk)],
            sems.at[i],
        )
        for i in range(_N_CHUNKS)
    ]
    for c in copies:
        c.start()
    for c in copies:
        c.wait()


def kernel(x, pe_table):
    del x
    max_pos, emb_dim = pe_table.shape
    out = pl.pallas_call(
        _dma_body,
        in_specs=[pl.BlockSpec(memory_space=pltpu.MemorySpace.HBM)],
        out_specs=pl.BlockSpec(memory_space=pltpu.MemorySpace.HBM),
        out_shape=jax.ShapeDtypeStruct((max_pos, emb_dim), pe_table.dtype),
        scratch_shapes=[pltpu.SemaphoreType.DMA((_N_CHUNKS,))],
    )(pe_table)
    return out[None]


# speedup vs baseline: 41.3495x; 41.3495x over previous
import jax
import jax.numpy as jnp
from jax.experimental import pallas as pl


def _copy_body(pe_ref, o_ref):
    o_ref[...] = pe_ref[...]


def kernel(x, pe_table):
    del x
    max_pos, emb_dim = pe_table.shape
    blk = 512
    out = pl.pallas_call(
        _copy_body,
        grid=(max_pos // blk,),
        in_specs=[pl.BlockSpec((blk, emb_dim), lambda i: (i, 0))],
        out_specs=pl.BlockSpec((blk, emb_dim), lambda i: (i, 0)),
        out_shape=jax.ShapeDtypeStruct((max_pos, emb_dim), pe_table.dtype),
    )(pe_table)
    return out[None]


# speedup vs baseline: 48.6181x; 1.1758x over previous
import jax
import jax.numpy as jnp
from jax.experimental import pallas as pl


def _copy_body(pe_ref, o_ref):
    o_ref[...] = pe_ref[...]


def kernel(x, pe_table):
    del x
    max_pos, emb_dim = pe_table.shape
    blk = 2048
    out = pl.pallas_call(
        _copy_body,
        grid=(max_pos // blk,),
        in_specs=[pl.BlockSpec((blk, emb_dim), lambda i: (i, 0))],
        out_specs=pl.BlockSpec((blk, emb_dim), lambda i: (i, 0)),
        out_shape=jax.ShapeDtypeStruct((max_pos, emb_dim), pe_table.dtype),
    )(pe_table)
    return out[None]
